# baseline (speedup 1.0000x reference)
import functools

import jax
import jax.numpy as jnp
from jax import lax
from jax.experimental import pallas as pl
from jax.experimental.pallas import tpu as pltpu
from jax.experimental.pallas import tpu_sc as plsc

_K = 8192
_D = 64
_T = 16384
_R = 2048
_GRID = _T // _R
_COMMITMENT = 0.25
_IDX_CHUNK = 128


_CHUNK = 4096
_SUB = 128


def _dist_argmin_body(emb2_ref, x_ref, e2_ref, x2_ref, idx_ref, loss_ref):
    i = pl.program_id(0)
    xb = x_ref[...].astype(jnp.bfloat16)
    x2b8 = jnp.broadcast_to(x2_ref[0], (8, _R))
    s_iota = lax.broadcasted_iota(jnp.int32, (8, _R), 0).astype(jnp.float32)
    acc_b = aidx = gmin = None
    for c in range(_K // _CHUNK):
        m8 = k8 = None
        for b in range(_CHUNK // _SUB):
            off = c * _CHUNK + b * _SUB
            eblk = emb2_ref[pl.ds(off, _SUB), :]
            mm2 = lax.dot_general(
                eblk, xb, (((1,), (1,)), ((), ())),
                preferred_element_type=jnp.float32)
            for k in range(_SUB // 8):
                sl = off + 8 * k
                dk = (e2_ref[pl.ds(sl, 8), :] + x2b8) - mm2[8 * k:8 * k + 8, :]
                kf = float(b * (_SUB // 8) + k)
                if b == 0 and k == 0:
                    m8 = dk
                    k8 = jnp.zeros((8, _R), jnp.float32)
                else:
                    u = dk < m8
                    k8 = jnp.where(u, kf, k8)
                    m8 = jnp.minimum(m8, dk)
        widx8 = k8 * 8.0 + s_iota + float(c * _CHUNK)
        wmin = jnp.min(m8, axis=0, keepdims=True)
        widx = jnp.min(jnp.where(m8 == wmin, widx8, float(_K)),
                       axis=0, keepdims=True)
        wmin_b = wmin.astype(jnp.bfloat16).astype(jnp.float32)
        if c == 0:
            acc_b, aidx, gmin = wmin_b, widx, wmin
        else:
            upd = wmin < acc_b
            aidx = jnp.where(upd, widx, aidx)
            acc_b = jnp.where(upd, wmin_b, acc_b)
            gmin = jnp.minimum(gmin, wmin)
    idx_ref[...] = aidx.astype(jnp.int32).reshape(1, 1, _R)

    @pl.when(i == 0)
    def _():
        loss_ref[0, 0] = 0.0

    loss_ref[0, 0] += jnp.sum(gmin)


def _tc_distance_argmin(flat, embeddings_x2, e2_col, x2_tiles):
    return pl.pallas_call(
        _dist_argmin_body,
        grid=(_GRID,),
        in_specs=[
            pl.BlockSpec((_K, _D), lambda i: (0, 0)),
            pl.BlockSpec((_R, _D), lambda i: (i, 0)),
            pl.BlockSpec((_K, 1), lambda i: (0, 0)),
            pl.BlockSpec((1, 1, _R), lambda i: (i, 0, 0)),
        ],
        out_specs=[
            pl.BlockSpec((1, 1, _R), lambda i: (i, 0, 0)),
            pl.BlockSpec((1, 1), lambda i: (0, 0), memory_space=pltpu.SMEM),
        ],
        out_shape=[
            jax.ShapeDtypeStruct((_GRID, 1, _R), jnp.int32),
            jax.ShapeDtypeStruct((1, 1), jnp.float32),
        ],
    )(embeddings_x2, flat, e2_col, x2_tiles)


def _sc_gather(table_padded, idx3d):
    info = plsc.get_sparse_core_info()
    n_workers = info.num_cores * info.num_subcores
    rows_per_w = _T // n_workers
    chunks = rows_per_w // _IDX_CHUNK

    mesh = plsc.VectorSubcoreMesh(core_axis_name="c", subcore_axis_name="s")

    @functools.partial(
        pl.kernel, mesh=mesh,
        out_type=jax.ShapeDtypeStruct((_T, 2 * _D), jnp.float32),
        scratch_types=[
            pltpu.VMEM((chunks, _IDX_CHUNK), jnp.int32),
            pltpu.VMEM((rows_per_w, 2 * _D), jnp.float32),
            pltpu.SemaphoreType.DMA,
        ],
    )
    def gather_kernel(table_hbm, idx_hbm, out_hbm, idx_v, rows_v, sem):
        wid = lax.axis_index("s") * info.num_cores + lax.axis_index("c")
        pltpu.sync_copy(idx_hbm.at[wid], idx_v)
        copies = [
            pltpu.async_copy(table_hbm.at[idx_v.at[j]],
                             rows_v.at[pl.ds(j * _IDX_CHUNK, _IDX_CHUNK)],
                             sem)
            for j in range(chunks)
        ]
        for c in copies:
            c.wait()
        pltpu.sync_copy(rows_v, out_hbm.at[pl.ds(wid * rows_per_w, rows_per_w)])

    return gather_kernel(table_padded, idx3d)


def kernel(inputs, embeddings):
    flat = lax.stop_gradient(inputs).reshape(-1, _D)
    e2 = jnp.sum(embeddings ** 2, axis=1)
    x2 = jnp.sum(flat ** 2, axis=1, keepdims=True)
    idx_tiles, loss_sum = _tc_distance_argmin(
        flat, embeddings * 2.0,
        e2.reshape(_K, 1), x2.reshape(_GRID, 1, _R))
    table_padded = jnp.pad(embeddings, ((0, 0), (0, _D)))
    gathered = _sc_gather(table_padded,
                          idx_tiles.reshape(-1, 4, _IDX_CHUNK))
    quantized = gathered[:, :_D]
    loss = (loss_sum[0, 0] / (_T * _D)) * (1.0 + _COMMITMENT)
    return (loss, quantized.reshape(inputs.shape))


# speedup vs baseline: 1.0291x; 1.0291x over previous
import functools

import jax
import jax.numpy as jnp
from jax import lax
from jax.experimental import pallas as pl
from jax.experimental.pallas import tpu as pltpu
from jax.experimental.pallas import tpu_sc as plsc

_K = 8192
_D = 64
_T = 16384
_R = 1024
_GRID = _T // _R
_COMMITMENT = 0.25
_IDX_CHUNK = 128


_CHUNK = 4096
_SUB = 128


def _dist_argmin_body(emb2_ref, x_ref, e2_ref, x2_ref, idx_ref, loss_ref):
    i = pl.program_id(0)
    xb = x_ref[...].astype(jnp.bfloat16)
    x2b8 = jnp.broadcast_to(x2_ref[0], (8, _R))
    s_iota = lax.broadcasted_iota(jnp.int32, (8, _R), 0).astype(jnp.float32)
    acc_b = aidx = gmin = None
    for c in range(_K // _CHUNK):
        m8 = k8 = None
        for b in range(_CHUNK // _SUB):
            off = c * _CHUNK + b * _SUB
            eblk = emb2_ref[pl.ds(off, _SUB), :]
            mm2 = lax.dot_general(
                eblk, xb, (((1,), (1,)), ((), ())),
                preferred_element_type=jnp.float32)
            for k in range(_SUB // 8):
                sl = off + 8 * k
                dk = (e2_ref[pl.ds(sl, 8), :] + x2b8) - mm2[8 * k:8 * k + 8, :]
                kf = float(b * (_SUB // 8) + k)
                if b == 0 and k == 0:
                    m8 = dk
                    k8 = jnp.zeros((8, _R), jnp.float32)
                else:
                    u = dk < m8
                    k8 = jnp.where(u, kf, k8)
                    m8 = jnp.minimum(m8, dk)
        widx8 = k8 * 8.0 + s_iota + float(c * _CHUNK)
        wmin = jnp.min(m8, axis=0, keepdims=True)
        widx = jnp.min(jnp.where(m8 == wmin, widx8, float(_K)),
                       axis=0, keepdims=True)
        wmin_b = wmin.astype(jnp.bfloat16).astype(jnp.float32)
        if c == 0:
            acc_b, aidx, gmin = wmin_b, widx, wmin
        else:
            upd = wmin < acc_b
            aidx = jnp.where(upd, widx, aidx)
            acc_b = jnp.where(upd, wmin_b, acc_b)
            gmin = jnp.minimum(gmin, wmin)
    idx_ref[...] = aidx.astype(jnp.int32).reshape(1, 1, _R)

    @pl.when(i == 0)
    def _():
        loss_ref[0, 0] = 0.0

    loss_ref[0, 0] += jnp.sum(gmin)


def _tc_distance_argmin(flat, embeddings_x2, e2_col, x2_tiles):
    return pl.pallas_call(
        _dist_argmin_body,
        grid=(_GRID,),
        in_specs=[
            pl.BlockSpec((_K, _D), lambda i: (0, 0)),
            pl.BlockSpec((_R, _D), lambda i: (i, 0)),
            pl.BlockSpec((_K, 1), lambda i: (0, 0)),
            pl.BlockSpec((1, 1, _R), lambda i: (i, 0, 0)),
        ],
        out_specs=[
            pl.BlockSpec((1, 1, _R), lambda i: (i, 0, 0)),
            pl.BlockSpec((1, 1), lambda i: (0, 0), memory_space=pltpu.SMEM),
        ],
        out_shape=[
            jax.ShapeDtypeStruct((_GRID, 1, _R), jnp.int32),
            jax.ShapeDtypeStruct((1, 1), jnp.float32),
        ],
    )(embeddings_x2, flat, e2_col, x2_tiles)


def _sc_gather(table_padded, idx3d):
    info = plsc.get_sparse_core_info()
    n_workers = info.num_cores * info.num_subcores
    rows_per_w = _T // n_workers
    chunks = rows_per_w // _IDX_CHUNK

    mesh = plsc.VectorSubcoreMesh(core_axis_name="c", subcore_axis_name="s")

    @functools.partial(
        pl.kernel, mesh=mesh,
        out_type=jax.ShapeDtypeStruct((_T, 2 * _D), jnp.float32),
        scratch_types=[
            pltpu.VMEM((chunks, _IDX_CHUNK), jnp.int32),
            pltpu.VMEM((rows_per_w, 2 * _D), jnp.float32),
            pltpu.SemaphoreType.DMA,
        ],
    )
    def gather_kernel(table_hbm, idx_hbm, out_hbm, idx_v, rows_v, sem):
        wid = lax.axis_index("s") * info.num_cores + lax.axis_index("c")
        pltpu.sync_copy(idx_hbm.at[wid], idx_v)
        copies = [
            pltpu.async_copy(table_hbm.at[idx_v.at[j]],
                             rows_v.at[pl.ds(j * _IDX_CHUNK, _IDX_CHUNK)],
                             sem)
            for j in range(chunks)
        ]
        for c in copies:
            c.wait()
        pltpu.sync_copy(rows_v, out_hbm.at[pl.ds(wid * rows_per_w, rows_per_w)])

    return gather_kernel(table_padded, idx3d)


def kernel(inputs, embeddings):
    flat = lax.stop_gradient(inputs).reshape(-1, _D)
    e2 = jnp.sum(embeddings ** 2, axis=1)
    x2 = jnp.sum(flat ** 2, axis=1, keepdims=True)
    idx_tiles, loss_sum = _tc_distance_argmin(
        flat, embeddings * 2.0,
        e2.reshape(_K, 1), x2.reshape(_GRID, 1, _R))
    table_padded = jnp.pad(embeddings, ((0, 0), (0, _D)))
    gathered = _sc_gather(table_padded,
                          idx_tiles.reshape(-1, 4, _IDX_CHUNK))
    quantized = gathered[:, :_D]
    loss = (loss_sum[0, 0] / (_T * _D)) * (1.0 + _COMMITMENT)
    return (loss, quantized.reshape(inputs.shape))


# speedup vs baseline: 1.0567x; 1.0268x over previous
import functools

import jax
import jax.numpy as jnp
from jax import lax
from jax.experimental import pallas as pl
from jax.experimental.pallas import tpu as pltpu
from jax.experimental.pallas import tpu_sc as plsc

_K = 8192
_D = 64
_T = 16384
_R = 1024
_GRID = _T // _R
_COMMITMENT = 0.25
_IDX_CHUNK = 128


_CHUNK = 4096
_SUB = 128


def _dist_argmin_body(emb2_ref, x_ref, e2_ref, x2_ref, idx_ref, loss_ref):
    i = pl.program_id(0)
    xb = x_ref[...].astype(jnp.bfloat16)
    x2b8 = jnp.broadcast_to(x2_ref[0], (8, _R))
    s_iota = lax.broadcasted_iota(jnp.int32, (8, _R), 0).astype(jnp.float32)
    acc_b = aidx = gmin = None
    for c in range(_K // _CHUNK):
        m8 = k8 = None
        for b in range(_CHUNK // _SUB):
            off = c * _CHUNK + b * _SUB
            eblk = emb2_ref[pl.ds(off, _SUB), :]
            mm2 = lax.dot_general(
                eblk, xb, (((1,), (0,)), ((), ())),
                preferred_element_type=jnp.float32)
            for k in range(_SUB // 8):
                sl = off + 8 * k
                dk = (e2_ref[pl.ds(sl, 8), :] + x2b8) - mm2[8 * k:8 * k + 8, :]
                kf = float(b * (_SUB // 8) + k)
                if b == 0 and k == 0:
                    m8 = dk
                    k8 = jnp.zeros((8, _R), jnp.float32)
                else:
                    u = dk < m8
                    k8 = jnp.where(u, kf, k8)
                    m8 = jnp.minimum(m8, dk)
        widx8 = k8 * 8.0 + s_iota + float(c * _CHUNK)
        wmin = jnp.min(m8, axis=0, keepdims=True)
        widx = jnp.min(jnp.where(m8 == wmin, widx8, float(_K)),
                       axis=0, keepdims=True)
        wmin_b = wmin.astype(jnp.bfloat16).astype(jnp.float32)
        if c == 0:
            acc_b, aidx, gmin = wmin_b, widx, wmin
        else:
            upd = wmin < acc_b
            aidx = jnp.where(upd, widx, aidx)
            acc_b = jnp.where(upd, wmin_b, acc_b)
            gmin = jnp.minimum(gmin, wmin)
    idx_ref[...] = aidx.astype(jnp.int32).reshape(1, 1, _R)

    @pl.when(i == 0)
    def _():
        loss_ref[0, 0] = 0.0

    loss_ref[0, 0] += jnp.sum(gmin)


def _tc_distance_argmin(flat_t, embeddings_x2, e2_col, x2_tiles):
    return pl.pallas_call(
        _dist_argmin_body,
        grid=(_GRID,),
        in_specs=[
            pl.BlockSpec((_K, _D), lambda i: (0, 0)),
            pl.BlockSpec((_D, _R), lambda i: (0, i)),
            pl.BlockSpec((_K, 1), lambda i: (0, 0)),
            pl.BlockSpec((1, 1, _R), lambda i: (i, 0, 0)),
        ],
        out_specs=[
            pl.BlockSpec((1, 1, _R), lambda i: (i, 0, 0)),
            pl.BlockSpec((1, 1), lambda i: (0, 0), memory_space=pltpu.SMEM),
        ],
        out_shape=[
            jax.ShapeDtypeStruct((_GRID, 1, _R), jnp.int32),
            jax.ShapeDtypeStruct((1, 1), jnp.float32),
        ],
    )(embeddings_x2, flat_t, e2_col, x2_tiles)


def _sc_gather(table_padded, idx3d):
    info = plsc.get_sparse_core_info()
    n_workers = info.num_cores * info.num_subcores
    rows_per_w = _T // n_workers
    chunks = rows_per_w // _IDX_CHUNK

    mesh = plsc.VectorSubcoreMesh(core_axis_name="c", subcore_axis_name="s")

    @functools.partial(
        pl.kernel, mesh=mesh,
        out_type=jax.ShapeDtypeStruct((_T, 2 * _D), jnp.float32),
        scratch_types=[
            pltpu.VMEM((chunks, _IDX_CHUNK), jnp.int32),
            pltpu.VMEM((rows_per_w, 2 * _D), jnp.float32),
            pltpu.SemaphoreType.DMA,
        ],
    )
    def gather_kernel(table_hbm, idx_hbm, out_hbm, idx_v, rows_v, sem):
        wid = lax.axis_index("s") * info.num_cores + lax.axis_index("c")
        pltpu.sync_copy(idx_hbm.at[wid], idx_v)
        copies = [
            pltpu.async_copy(table_hbm.at[idx_v.at[j]],
                             rows_v.at[pl.ds(j * _IDX_CHUNK, _IDX_CHUNK)],
                             sem)
            for j in range(chunks)
        ]
        for c in copies:
            c.wait()
        pltpu.sync_copy(rows_v, out_hbm.at[pl.ds(wid * rows_per_w, rows_per_w)])

    return gather_kernel(table_padded, idx3d)


def kernel(inputs, embeddings):
    flat = lax.stop_gradient(inputs).reshape(-1, _D)
    e2 = jnp.sum(embeddings ** 2, axis=1)
    x2 = jnp.sum(flat ** 2, axis=1, keepdims=True)
    idx_tiles, loss_sum = _tc_distance_argmin(
        flat.T, embeddings * 2.0,
        e2.reshape(_K, 1), x2.reshape(_GRID, 1, _R))
    table_padded = jnp.pad(embeddings, ((0, 0), (0, _D)))
    gathered = _sc_gather(table_padded,
                          idx_tiles.reshape(-1, 4, _IDX_CHUNK))
    quantized = gathered[:, :_D]
    loss = (loss_sum[0, 0] / (_T * _D)) * (1.0 + _COMMITMENT)
    return (loss, quantized.reshape(inputs.shape))
